# baseline (speedup 1.0000x reference)
import functools

import jax
import jax.numpy as jnp
from jax import lax
from jax.experimental import pallas as pl
from jax.experimental.pallas import tpu as pltpu
from jax.experimental.pallas import tpu_sc as plsc

_CHUNK = 2688
_FULL_CHUNKS = 372
_PER_WORKER = _FULL_CHUNKS // 4
_PAIRS = _PER_WORKER // 2
_TAIL_OFF = _FULL_CHUNKS * _CHUNK
_TAIL = 1000000 - _TAIL_OFF


def _sc_copy_body(src, dst, vbuf, shared, tail_buf,
                  in_a, in_b, out_a, out_b):
    c = lax.axis_index("c")
    s = lax.axis_index("s")
    wid = s * 2 + c
    band = wid // 4
    q = wid % 4
    rows = pl.ds(band * 8, 8)

    def cols(j):
        return pl.ds((q + 4 * j) * _CHUNK, _CHUNK)

    def step(p, _):
        slot = p % 2

        @pl.when(p < _PAIRS)
        def _():
            @pl.when(p >= 2)
            def _():
                pltpu.make_async_copy(vbuf.at[slot], dst.at[rows, cols(2 * (p - 2))],
                                      out_a.at[slot]).wait()
                pltpu.make_async_copy(shared.at[s, slot], dst.at[rows, cols(2 * (p - 2) + 1)],
                                      out_b.at[slot]).wait()

            pltpu.make_async_copy(src.at[rows, cols(2 * p)], vbuf.at[slot],
                                  in_a.at[slot]).start()
            pltpu.make_async_copy(src.at[rows, cols(2 * p + 1)], shared.at[s, slot],
                                  in_b.at[slot]).start()

        @pl.when(p >= 1)
        def _():
            s1 = (p - 1) % 2
            pltpu.make_async_copy(src.at[rows, cols(2 * (p - 1))], vbuf.at[s1],
                                  in_a.at[s1]).wait()
            pltpu.make_async_copy(vbuf.at[s1], dst.at[rows, cols(2 * (p - 1))],
                                  out_a.at[s1]).start()
            pltpu.make_async_copy(src.at[rows, cols(2 * (p - 1) + 1)], shared.at[s, s1],
                                  in_b.at[s1]).wait()
            pltpu.make_async_copy(shared.at[s, s1], dst.at[rows, cols(2 * (p - 1) + 1)],
                                  out_b.at[s1]).start()

        return _

    lax.fori_loop(0, _PAIRS + 1, step, None)

    def drain(p, _):
        slot = p % 2
        pltpu.make_async_copy(vbuf.at[slot], dst.at[rows, cols(2 * p)],
                              out_a.at[slot]).wait()
        pltpu.make_async_copy(shared.at[s, slot], dst.at[rows, cols(2 * p + 1)],
                              out_b.at[slot]).wait()
        return _

    lax.fori_loop(_PAIRS - 2, _PAIRS, drain, None)

    lcols = cols(_PER_WORKER - 1)
    pltpu.sync_copy(src.at[rows, lcols], vbuf.at[0])
    pltpu.sync_copy(vbuf.at[0], dst.at[rows, lcols])

    @pl.when(q == 0)
    def _():
        tcols = pl.ds(_TAIL_OFF, _TAIL)
        pltpu.sync_copy(src.at[rows, tcols], tail_buf)
        pltpu.sync_copy(tail_buf, dst.at[rows, tcols])


def kernel(inputs, p_emb):
    n, d = p_emb.shape
    t = p_emb.T
    mesh = plsc.VectorSubcoreMesh(core_axis_name="c", subcore_axis_name="s")
    sc_copy = functools.partial(
        pl.kernel,
        mesh=mesh,
        out_type=jax.ShapeDtypeStruct(t.shape, t.dtype),
        scratch_types=[
            pltpu.VMEM((2, 8, _CHUNK), jnp.float32),
            pltpu.VMEM_SHARED((16, 2, 8, _CHUNK), jnp.float32),
            pltpu.VMEM((8, _TAIL), jnp.float32),
            pltpu.SemaphoreType.DMA((2,)),
            pltpu.SemaphoreType.DMA((2,)),
            pltpu.SemaphoreType.DMA((2,)),
            pltpu.SemaphoreType.DMA((2,)),
        ],
    )(_sc_copy_body)
    return sc_copy(t).T


# speedup vs baseline: 1.1894x; 1.1894x over previous
import jax
import jax.numpy as jnp
from jax.experimental import pallas as pl
from jax.experimental.pallas import tpu as pltpu

_BLOCK_COLS = 32768


def _copy_block(in_ref, out_ref):
    out_ref[...] = in_ref[...]


def kernel(inputs, p_emb):
    n, d = p_emb.shape
    t = p_emb.T
    grid = pl.cdiv(n, _BLOCK_COLS)
    out = pl.pallas_call(
        _copy_block,
        grid=(grid,),
        in_specs=[pl.BlockSpec((d, _BLOCK_COLS), lambda i: (0, i))],
        out_specs=pl.BlockSpec((d, _BLOCK_COLS), lambda i: (0, i)),
        out_shape=jax.ShapeDtypeStruct(t.shape, t.dtype),
    )(t)
    return out.T
